# baseline (speedup 1.0000x reference)
import functools

import jax
import jax.numpy as jnp
from jax import lax
from jax.experimental import pallas as pl
from jax.experimental.pallas import tpu as pltpu
from jax.experimental.pallas import tpu_sc as plsc

NNZ = 250000
NUM_USERS = 16384
NUM_ITEMS = 100000
H0 = 256
H1 = 128

NC = 2
NS = 16
NW = NC * NS
HP = 4
HC = H0 // HP
UPT = NUM_USERS // NS

EB = 128
ENB = 62
DB = 64
DNB = 124
NNZ_PAD = NW * ENB * EB


def _enc_body(item_h, user_h, rat_h, w4_h, out_h,
              item_v, user_v, rat_v, gbuf, sbuf, zbuf, acc_sh, sems, ssems):
    c = lax.axis_index("c")
    s = lax.axis_index("s")
    w = c * NS + s
    pltpu.sync_copy(item_h.at[w], item_v)
    pltpu.sync_copy(user_h.at[w], user_v)
    pltpu.sync_copy(rat_h.at[w], rat_v)

    def zrow(r, carry):
        for k in range(HC // 16):
            zbuf[r, pl.ds(k * 16, 16)] = jnp.zeros((16,), jnp.float32)
        return carry
    lax.fori_loop(0, 128, zrow, 0)

    for j in range(HP):
        def issue(b, p):
            return pltpu.async_copy(w4_h.at[j].at[item_v.at[b]],
                                    gbuf.at[p], sems.at[p])

        def wait(b, p):
            pltpu.make_async_copy(w4_h.at[j].at[item_v.at[b]],
                                  gbuf.at[p], sems.at[p]).wait()

        def scale_scatter(b, p):
            @pl.when(b >= 2)
            def _():
                pltpu.make_async_copy(sbuf.at[p], acc_sh.at[user_v.at[b]],
                                      ssems.at[p]).wait()

            def grp(g, carry2):
                rv = rat_v[b, pl.ds(g * 16, 16)]
                for l in range(16):
                    rt = rv[l]
                    r = g * 16 + l
                    for k in range(HC // 16):
                        sbuf[p, r, pl.ds(k * 16, 16)] = (
                            gbuf[p, r, pl.ds(k * 16, 16)] * rt)
                return carry2
            lax.fori_loop(0, EB // 16, grp, 0)
            pltpu.async_copy(sbuf.at[p], acc_sh.at[user_v.at[b]],
                             ssems.at[p], add=True)

        def zcopy(b, carry):
            pltpu.sync_copy(zbuf, acc_sh.at[pl.ds(s * UPT + b * 128, 128)])
            return carry
        lax.fori_loop(0, UPT // 128, zcopy, 0)
        plsc.subcore_barrier()

        issue(0, 0)

        def pair(i, carry):
            b0 = 2 * i
            issue(b0 + 1, 1)
            wait(b0, 0)
            scale_scatter(b0, 0)

            @pl.when(i < ENB // 2 - 1)
            def _():
                issue(b0 + 2, 0)
            wait(b0 + 1, 1)
            scale_scatter(b0 + 1, 1)
            return carry
        lax.fori_loop(0, ENB // 2, pair, 0)
        for p in range(2):
            pltpu.make_async_copy(sbuf.at[p],
                                  acc_sh.at[user_v.at[ENB - 2 + p]],
                                  ssems.at[p]).wait()

        plsc.subcore_barrier()
        pltpu.sync_copy(acc_sh.at[pl.ds(s * UPT, UPT)],
                        out_h.at[c, j, pl.ds(s * UPT, UPT)])
        plsc.subcore_barrier()


_enc_call = functools.partial(
    pl.kernel,
    out_type=jax.ShapeDtypeStruct((NC, HP, NUM_USERS, HC), jnp.float32),
    mesh=plsc.VectorSubcoreMesh(core_axis_name="c", subcore_axis_name="s",
                                num_cores=NC, num_subcores=NS),
    scratch_types=[
        pltpu.VMEM((ENB, EB), jnp.int32),
        pltpu.VMEM((ENB, EB), jnp.int32),
        pltpu.VMEM((ENB, EB), jnp.float32),
        pltpu.VMEM((2, EB, HC), jnp.float32),
        pltpu.VMEM((2, EB, HC), jnp.float32),
        pltpu.VMEM((128, HC), jnp.float32),
        pltpu.VMEM_SHARED((NUM_USERS, HC), jnp.float32),
        pltpu.SemaphoreType.DMA((2,)),
        pltpu.SemaphoreType.DMA((2,)),
    ],
    compiler_params=pltpu.CompilerParams(use_tc_tiling_on_sc=False),
)(_enc_body)


def _mlp_body(p_ref, be_ref, w1_ref, b1_ref, w2_ref, b2_ref, out_ref):
    x = jnp.concatenate([p_ref[0, k] + p_ref[1, k] for k in range(HP)], axis=-1)
    h = jnp.tanh(x + be_ref[...])
    e = jnp.tanh(lax.dot_general(h, w1_ref[...], (((1,), (1,)), ((), ())),
                                 preferred_element_type=jnp.float32) + b1_ref[...])
    d = jnp.tanh(lax.dot_general(e, w2_ref[...], (((1,), (1,)), ((), ())),
                                 preferred_element_type=jnp.float32) + b2_ref[...])
    out_ref[...] = d


RB = 1024


def _mlp_call(parts, b_enc, W1, b1, W2, b2):
    return pl.pallas_call(
        _mlp_body,
        grid=(NUM_USERS // RB,),
        in_specs=[
            pl.BlockSpec((NC, HP, RB, HC), lambda i: (0, 0, i, 0)),
            pl.BlockSpec((1, H0), lambda i: (0, 0)),
            pl.BlockSpec((H1, H0), lambda i: (0, 0)),
            pl.BlockSpec((1, H1), lambda i: (0, 0)),
            pl.BlockSpec((H0, H1), lambda i: (0, 0)),
            pl.BlockSpec((1, H0), lambda i: (0, 0)),
        ],
        out_specs=pl.BlockSpec((RB, H0), lambda i: (i, 0)),
        out_shape=jax.ShapeDtypeStruct((NUM_USERS, H0), jnp.float32),
    )(parts, b_enc.reshape(1, H0), W1, b1.reshape(1, H1), W2, b2.reshape(1, H0))


NQ = NNZ_PAD // NW // 128


def _bias_body(b16_h, ti_h, out_h, ti_v, hi_v, bvg, bv_v, sems):
    c = lax.axis_index("c")
    s = lax.axis_index("s")
    w = c * NS + s
    pltpu.sync_copy(ti_h.at[w], ti_v)
    lanes = lax.iota(jnp.int32, 16)

    def issue(b, p):
        for k in range(8):
            hi_v[p, pl.ds(k * 16, 16)] = lax.shift_right_logical(
                ti_v[b, pl.ds(k * 16, 16)], 4)
        pltpu.async_copy(b16_h.at[hi_v.at[p]], bvg.at[p], sems.at[p])

    def wait(p):
        pltpu.make_async_copy(b16_h.at[hi_v.at[p]], bvg.at[p],
                              sems.at[p]).wait()

    def compute(b, p):
        for k in range(8):
            tl = jnp.bitwise_and(ti_v[b, pl.ds(k * 16, 16)],
                                 jnp.full((16,), 15, jnp.int32))
            bv_v[b, pl.ds(k * 16, 16)] = plsc.load_gather(
                bvg.at[p], [k * 16 + lanes, tl])

    issue(0, 0)

    def pair(i, carry):
        b0 = 2 * i
        issue(b0 + 1, 1)
        wait(0)
        compute(b0, 0)

        @pl.when(i < NQ // 2 - 1)
        def _():
            issue(b0 + 2, 0)
        wait(1)
        compute(b0 + 1, 1)
        return carry
    lax.fori_loop(0, NQ // 2, pair, 0)
    pltpu.sync_copy(bv_v, out_h.at[w])


_bias_call = functools.partial(
    pl.kernel,
    out_type=jax.ShapeDtypeStruct((NW, NQ, 128), jnp.float32),
    mesh=plsc.VectorSubcoreMesh(core_axis_name="c", subcore_axis_name="s",
                                num_cores=NC, num_subcores=NS),
    scratch_types=[
        pltpu.VMEM((NQ, 128), jnp.int32),
        pltpu.VMEM((2, 128), jnp.int32),
        pltpu.VMEM((2, 128, 16), jnp.float32),
        pltpu.VMEM((NQ, 128), jnp.float32),
        pltpu.SemaphoreType.DMA((2,)),
    ],
    compiler_params=pltpu.CompilerParams(use_tc_tiling_on_sc=False,
                                         needs_layout_passes=False),
)(_bias_body)


NCH = NNZ_PAD // NW // 16
NRING = 4


def _dec_body(dec_h, wdec_h, bv_h, tu_h, ti_h, out_h,
              tu_v, ti_v, bv_v, bufU, bufI, pred_v, tbuf, sems):
    c = lax.axis_index("c")
    s = lax.axis_index("s")
    w = c * NS + s
    pltpu.sync_copy(tu_h.at[w], tu_v)
    pltpu.sync_copy(ti_h.at[w], ti_v)
    pltpu.sync_copy(bv_h.at[w], bv_v)
    lanes = lax.iota(jnp.int32, 16)

    def idx16(ref, q):
        return plsc.load_gather(ref, [jnp.full((16,), 0, jnp.int32) + q // 8,
                                      (q % 8) * 16 + lanes])

    def issue(q, k):
        pltpu.async_copy(dec_h.at[idx16(tu_v, q)], bufU.at[k], sems.at[k])
        pltpu.async_copy(wdec_h.at[idx16(ti_v, q)], bufI.at[k], sems.at[k])

    def wait(q, k):
        pltpu.make_async_copy(dec_h.at[idx16(tu_v, q)], bufU.at[k],
                              sems.at[k]).wait()
        pltpu.make_async_copy(wdec_h.at[idx16(ti_v, q)], bufI.at[k],
                              sems.at[k]).wait()

    def compute(q, k):
        for l in range(16):
            prods = [bufU[k, l, pl.ds(j * 16, 16)]
                     * bufI[k, l, pl.ds(j * 16, 16)]
                     for j in range(H0 // 16)]
            while len(prods) > 1:
                prods = [prods[i] + prods[i + 1]
                         for i in range(0, len(prods), 2)]
            plsc.store_scatter(tbuf, [lanes * 16 + l], prods[0])
        terms = [tbuf[pl.ds(j * 16, 16)] for j in range(16)]
        while len(terms) > 1:
            terms = [terms[i] + terms[i + 1] for i in range(0, len(terms), 2)]
        plsc.store_scatter(pred_v,
                           [jnp.full((16,), 0, jnp.int32) + q // 8,
                            (q % 8) * 16 + lanes],
                           terms[0] + idx16(bv_v, q))

    for k in range(NRING):
        issue(k, k)

    def step(i, carry):
        for k in range(NRING):
            q = NRING * i + k
            wait(q, k)
            compute(q, k)

            @pl.when(q + NRING < NCH)
            def _():
                issue(q + NRING, k)
        return carry
    lax.fori_loop(0, NCH // NRING, step, 0)
    pltpu.sync_copy(pred_v, out_h.at[w])


_dec_call = functools.partial(
    pl.kernel,
    out_type=jax.ShapeDtypeStruct((NW, NQ, 128), jnp.float32),
    mesh=plsc.VectorSubcoreMesh(core_axis_name="c", subcore_axis_name="s",
                                num_cores=NC, num_subcores=NS),
    scratch_types=[
        pltpu.VMEM((NQ, 128), jnp.int32),
        pltpu.VMEM((NQ, 128), jnp.int32),
        pltpu.VMEM((NQ, 128), jnp.float32),
        pltpu.VMEM((NRING, 16, H0), jnp.float32),
        pltpu.VMEM((NRING, 16, H0), jnp.float32),
        pltpu.VMEM((NQ, 128), jnp.float32),
        pltpu.VMEM((256,), jnp.float32),
        pltpu.SemaphoreType.DMA((NRING,)),
    ],
    compiler_params=pltpu.CompilerParams(use_tc_tiling_on_sc=True,
                                         needs_layout_passes=False),
)(_dec_body)


LR = NNZ_PAD // 128


def _loss_body(pred_ref, tr_ref, out_ref):
    rows = lax.broadcasted_iota(jnp.int32, (LR, 128), 0)
    cols = lax.broadcasted_iota(jnp.int32, (LR, 128), 1)
    valid = (rows * 128 + cols) < NNZ
    d = pred_ref[...] - tr_ref[...]
    sq = jnp.where(valid, d * d, 0.0)
    out_ref[...] = (jnp.sum(sq) / NNZ)[None, None]


def _loss_call(pred_p, tr_p):
    return pl.pallas_call(
        _loss_body,
        out_shape=jax.ShapeDtypeStruct((1, 1), jnp.float32),
    )(pred_p.reshape(LR, 128), tr_p.reshape(LR, 128))


def _pad_tiles(a, dtype, nb, bsz):
    a = a.astype(dtype)
    return jnp.pad(a, (0, NNZ_PAD - NNZ)).reshape(NW, nb, bsz)


def kernel(user, item, rating, target_user, target_item, target_rating,
           W_enc, b_enc, W1, b1, W2, b2, W_dec, b_dec):
    user_p = _pad_tiles(user, jnp.int32, ENB, EB)
    item_p = _pad_tiles(item, jnp.int32, ENB, EB)
    rat_p = _pad_tiles(rating, jnp.float32, ENB, EB)
    tu_p = _pad_tiles(target_user, jnp.int32, NQ, 128)
    ti_p = _pad_tiles(target_item, jnp.int32, NQ, 128)
    tr_p = _pad_tiles(target_rating, jnp.float32, NQ, 128)

    w4 = W_enc.reshape(HP, HC, NUM_ITEMS).transpose(0, 2, 1)

    b16 = b_dec.reshape(NUM_ITEMS // 16, 16)
    bv_all = _bias_call(b16, ti_p)
    parts = _enc_call(item_p, user_p, rat_p, w4)
    decoded = _mlp_call(parts, b_enc, W1, b1, W2, b2)
    pred_p = _dec_call(decoded, W_dec, bv_all, tu_p, ti_p)
    loss = _loss_call(pred_p, tr_p)[0, 0]
    pred = pred_p.reshape(-1)[:NNZ]
    return pred, loss


# speedup vs baseline: 1.0852x; 1.0852x over previous
import functools

import jax
import jax.numpy as jnp
from jax import lax
from jax.experimental import pallas as pl
from jax.experimental.pallas import tpu as pltpu
from jax.experimental.pallas import tpu_sc as plsc

NNZ = 250000
NUM_USERS = 16384
NUM_ITEMS = 100000
H0 = 256
H1 = 128

NC = 2
NS = 16
NW = NC * NS
HP = 4
HC = H0 // HP
UPT = NUM_USERS // NS

EB = 128
ENB = 62
DB = 64
DNB = 124
NNZ_PAD = NW * ENB * EB


def _enc_body(item_h, user_h, rat_h, w4_h, out_h,
              item_v, user_v, rat_v, gbuf, sbuf, zbuf, acc_sh, sems, ssems):
    c = lax.axis_index("c")
    s = lax.axis_index("s")
    w = c * NS + s
    pltpu.sync_copy(item_h.at[w], item_v)
    pltpu.sync_copy(user_h.at[w], user_v)
    pltpu.sync_copy(rat_h.at[w], rat_v)

    def zrow(r, carry):
        for k in range(HC // 16):
            zbuf[r, pl.ds(k * 16, 16)] = jnp.zeros((16,), jnp.float32)
        return carry
    lax.fori_loop(0, 128, zrow, 0)

    for j in range(HP):
        def issue(b, p):
            return pltpu.async_copy(w4_h.at[j].at[item_v.at[b]],
                                    gbuf.at[p], sems.at[p])

        def wait(b, p):
            pltpu.make_async_copy(w4_h.at[j].at[item_v.at[b]],
                                  gbuf.at[p], sems.at[p]).wait()

        def scale_scatter(b, p):
            @pl.when(b >= 2)
            def _():
                pltpu.make_async_copy(sbuf.at[p], acc_sh.at[user_v.at[b]],
                                      ssems.at[p]).wait()

            def grp(g, carry2):
                rv = rat_v[b, pl.ds(g * 16, 16)]
                for l in range(16):
                    rt = rv[l]
                    r = g * 16 + l
                    for k in range(HC // 16):
                        sbuf[p, r, pl.ds(k * 16, 16)] = (
                            gbuf[p, r, pl.ds(k * 16, 16)] * rt)
                return carry2
            lax.fori_loop(0, EB // 16, grp, 0)
            pltpu.async_copy(sbuf.at[p], acc_sh.at[user_v.at[b]],
                             ssems.at[p], add=True)

        def zcopy(b, carry):
            pltpu.sync_copy(zbuf, acc_sh.at[pl.ds(s * UPT + b * 128, 128)])
            return carry
        lax.fori_loop(0, UPT // 128, zcopy, 0)
        plsc.subcore_barrier()

        issue(0, 0)

        def pair(i, carry):
            b0 = 2 * i
            issue(b0 + 1, 1)
            wait(b0, 0)
            scale_scatter(b0, 0)

            @pl.when(i < ENB // 2 - 1)
            def _():
                issue(b0 + 2, 0)
            wait(b0 + 1, 1)
            scale_scatter(b0 + 1, 1)
            return carry
        lax.fori_loop(0, ENB // 2, pair, 0)
        for p in range(2):
            pltpu.make_async_copy(sbuf.at[p],
                                  acc_sh.at[user_v.at[ENB - 2 + p]],
                                  ssems.at[p]).wait()

        plsc.subcore_barrier()
        pltpu.sync_copy(acc_sh.at[pl.ds(s * UPT, UPT)],
                        out_h.at[c, j, pl.ds(s * UPT, UPT)])
        plsc.subcore_barrier()


_enc_call = functools.partial(
    pl.kernel,
    out_type=jax.ShapeDtypeStruct((NC, HP, NUM_USERS, HC), jnp.float32),
    mesh=plsc.VectorSubcoreMesh(core_axis_name="c", subcore_axis_name="s",
                                num_cores=NC, num_subcores=NS),
    scratch_types=[
        pltpu.VMEM((ENB, EB), jnp.int32),
        pltpu.VMEM((ENB, EB), jnp.int32),
        pltpu.VMEM((ENB, EB), jnp.float32),
        pltpu.VMEM((2, EB, HC), jnp.float32),
        pltpu.VMEM((2, EB, HC), jnp.float32),
        pltpu.VMEM((128, HC), jnp.float32),
        pltpu.VMEM_SHARED((NUM_USERS, HC), jnp.float32),
        pltpu.SemaphoreType.DMA((2,)),
        pltpu.SemaphoreType.DMA((2,)),
    ],
    compiler_params=pltpu.CompilerParams(use_tc_tiling_on_sc=False),
)(_enc_body)


def _mlp_body(p_ref, be_ref, w1_ref, b1_ref, w2_ref, b2_ref, out_ref):
    x = jnp.concatenate([p_ref[0, k] + p_ref[1, k] for k in range(HP)], axis=-1)
    h = jnp.tanh(x + be_ref[...])
    e = jnp.tanh(lax.dot_general(h, w1_ref[...], (((1,), (1,)), ((), ())),
                                 preferred_element_type=jnp.float32) + b1_ref[...])
    d = jnp.tanh(lax.dot_general(e, w2_ref[...], (((1,), (1,)), ((), ())),
                                 preferred_element_type=jnp.float32) + b2_ref[...])
    out_ref[...] = d


RB = 1024


def _mlp_call(parts, b_enc, W1, b1, W2, b2):
    return pl.pallas_call(
        _mlp_body,
        grid=(NUM_USERS // RB,),
        in_specs=[
            pl.BlockSpec((NC, HP, RB, HC), lambda i: (0, 0, i, 0)),
            pl.BlockSpec((1, H0), lambda i: (0, 0)),
            pl.BlockSpec((H1, H0), lambda i: (0, 0)),
            pl.BlockSpec((1, H1), lambda i: (0, 0)),
            pl.BlockSpec((H0, H1), lambda i: (0, 0)),
            pl.BlockSpec((1, H0), lambda i: (0, 0)),
        ],
        out_specs=pl.BlockSpec((RB, H0), lambda i: (i, 0)),
        out_shape=jax.ShapeDtypeStruct((NUM_USERS, H0), jnp.float32),
    )(parts, b_enc.reshape(1, H0), W1, b1.reshape(1, H1), W2, b2.reshape(1, H0))


NQ = NNZ_PAD // NW // 128


def _dec_body(dec_h, wdec_h, b16_h, tu_h, ti_h, out_h,
              tu_v, ti_v, bufU, bufI, bvbuf, hi_v, pred_v, tbuf, sems):
    c = lax.axis_index("c")
    s = lax.axis_index("s")
    w = c * NS + s
    pltpu.sync_copy(tu_h.at[w], tu_v)
    pltpu.sync_copy(ti_h.at[w], ti_v)
    lanes = lax.iota(jnp.int32, 16)

    def issue(b, p):
        for k in range(DB // 16):
            hi_v[p, pl.ds(k * 16, 16)] = lax.shift_right_logical(
                ti_v[b, pl.ds(k * 16, 16)], 4)
        pltpu.async_copy(dec_h.at[tu_v.at[b]], bufU.at[p], sems.at[p])
        pltpu.async_copy(wdec_h.at[ti_v.at[b]], bufI.at[p], sems.at[p])
        pltpu.async_copy(b16_h.at[hi_v.at[p]], bvbuf.at[p], sems.at[p])

    def wait(b, p):
        pltpu.make_async_copy(dec_h.at[tu_v.at[b]], bufU.at[p], sems.at[p]).wait()
        pltpu.make_async_copy(wdec_h.at[ti_v.at[b]], bufI.at[p], sems.at[p]).wait()
        pltpu.make_async_copy(b16_h.at[hi_v.at[p]], bvbuf.at[p], sems.at[p]).wait()

    def compute(b, p):
        def grp(g, carry):
            for l2 in range(8):
                accs = []
                for l in (2 * l2, 2 * l2 + 1):
                    r = g * 16 + l
                    prods = [bufU[p, r, pl.ds(j * 16, 16)]
                             * bufI[p, r, pl.ds(j * 16, 16)]
                             for j in range(H0 // 16)]
                    while len(prods) > 1:
                        prods = [prods[i] + prods[i + 1]
                                 for i in range(0, len(prods), 2)]
                    accs.append(prods[0])
                for l, acc in zip((2 * l2, 2 * l2 + 1), accs):
                    plsc.store_scatter(
                        tbuf, [lanes, jnp.full((16,), l, jnp.int32)], acc)
            terms = [tbuf[j, pl.ds(0, 16)] for j in range(16)]
            while len(terms) > 1:
                terms = [terms[i] + terms[i + 1]
                         for i in range(0, len(terms), 2)]
            tlow = jnp.bitwise_and(ti_v[b, pl.ds(g * 16, 16)],
                                   jnp.full((16,), 15, jnp.int32))
            bias = plsc.load_gather(bvbuf.at[p], [g * 16 + lanes, tlow])
            pred_v[b, pl.ds(g * 16, 16)] = terms[0] + bias
            return carry
        lax.fori_loop(0, DB // 16, grp, 0)

    issue(0, 0)

    def pair(i, carry):
        b0 = 2 * i
        issue(b0 + 1, 1)
        wait(b0, 0)
        compute(b0, 0)

        @pl.when(i < DNB // 2 - 1)
        def _():
            issue(b0 + 2, 0)
        wait(b0 + 1, 1)
        compute(b0 + 1, 1)
        return carry
    lax.fori_loop(0, DNB // 2, pair, 0)
    pltpu.sync_copy(pred_v, out_h.at[w])


_dec_call = functools.partial(
    pl.kernel,
    out_type=jax.ShapeDtypeStruct((NW, DNB, DB), jnp.float32),
    mesh=plsc.VectorSubcoreMesh(core_axis_name="c", subcore_axis_name="s",
                                num_cores=NC, num_subcores=NS),
    scratch_types=[
        pltpu.VMEM((DNB, DB), jnp.int32),
        pltpu.VMEM((DNB, DB), jnp.int32),
        pltpu.VMEM((2, DB, H0), jnp.float32),
        pltpu.VMEM((2, DB, H0), jnp.float32),
        pltpu.VMEM((2, DB, 16), jnp.float32),
        pltpu.VMEM((2, DB), jnp.int32),
        pltpu.VMEM((DNB, DB), jnp.float32),
        pltpu.VMEM((16, 16), jnp.float32),
        pltpu.SemaphoreType.DMA((2,)),
    ],
    compiler_params=pltpu.CompilerParams(use_tc_tiling_on_sc=False,
                                         needs_layout_passes=False),
)(_dec_body)


LR = NNZ_PAD // 128


def _loss_body(pred_ref, tr_ref, out_ref):
    rows = lax.broadcasted_iota(jnp.int32, (LR, 128), 0)
    cols = lax.broadcasted_iota(jnp.int32, (LR, 128), 1)
    valid = (rows * 128 + cols) < NNZ
    d = pred_ref[...] - tr_ref[...]
    sq = jnp.where(valid, d * d, 0.0)
    out_ref[...] = (jnp.sum(sq) / NNZ)[None, None]


def _loss_call(pred_p, tr_p):
    return pl.pallas_call(
        _loss_body,
        out_shape=jax.ShapeDtypeStruct((1, 1), jnp.float32),
    )(pred_p.reshape(LR, 128), tr_p.reshape(LR, 128))


def _pad_tiles(a, dtype, nb, bsz):
    a = a.astype(dtype)
    return jnp.pad(a, (0, NNZ_PAD - NNZ)).reshape(NW, nb, bsz)


def kernel(user, item, rating, target_user, target_item, target_rating,
           W_enc, b_enc, W1, b1, W2, b2, W_dec, b_dec):
    user_p = _pad_tiles(user, jnp.int32, ENB, EB)
    item_p = _pad_tiles(item, jnp.int32, ENB, EB)
    rat_p = _pad_tiles(rating, jnp.float32, ENB, EB)
    tu_p = _pad_tiles(target_user, jnp.int32, DNB, DB)
    ti_p = _pad_tiles(target_item, jnp.int32, DNB, DB)
    tr_p = _pad_tiles(target_rating, jnp.float32, DNB, DB)

    w4 = W_enc.reshape(HP, HC, NUM_ITEMS).transpose(0, 2, 1)

    b16 = b_dec.reshape(NUM_ITEMS // 16, 16)
    parts = _enc_call(item_p, user_p, rat_p, w4)
    decoded = _mlp_call(parts, b_enc, W1, b1, W2, b2)
    pred_p = _dec_call(decoded, W_dec, b16, tu_p, ti_p)
    loss = _loss_call(pred_p, tr_p)[0, 0]
    pred = pred_p.reshape(-1)[:NNZ]
    return pred, loss
